# baseline (speedup 1.0000x reference)
import functools

import jax
import jax.numpy as jnp
from jax import lax
from jax.experimental import pallas as pl
from jax.experimental.pallas import tpu as pltpu
from jax.experimental.pallas import tpu_sc as plsc

N = 10000
E = 320000
F_IN = 128
NC_OUT = 16
NG = 64

N_PAD = 10240
NUM_CORES = 2
NUM_SUBCORES = 16
NW = NUM_CORES * NUM_SUBCORES
EW = 10112
E_PAD = NW * EW
ROWS_PER_TILE = N_PAD // NUM_SUBCORES

_F32 = jnp.float32
_HIGH = jax.lax.Precision.HIGHEST


def _mesh():
    return plsc.VectorSubcoreMesh(
        core_axis_name="c", subcore_axis_name="s",
        num_cores=NUM_CORES, num_subcores=NUM_SUBCORES)


def _make_sc_scatter(D):
    SEG = D * N_PAD // NUM_SUBCORES

    @functools.partial(
        pl.kernel,
        out_type=jax.ShapeDtypeStruct((NUM_CORES, D * N_PAD), _F32),
        mesh=_mesh(),
        scratch_types=[
            pltpu.VMEM((EW,), jnp.int32),
            pltpu.VMEM((EW,), jnp.int32),
            pltpu.VMEM((EW,), _F32),
            pltpu.VMEM_SHARED((D * N_PAD,), _F32),
            pltpu.VMEM_SHARED((D * N_PAD,), _F32),
            pltpu.SemaphoreType.DMA,
        ],
    )
    def sc_scatter(gt_hbm, src_hbm, dst_hbm, out_hbm,
                   sidx, didx, vals, sh_g, sh_agg, sem):
        c = lax.axis_index("c")
        s = lax.axis_index("s")
        wid = s * NUM_CORES + c
        seg_sl = pl.ds(s * SEG, SEG)
        pltpu.sync_copy(src_hbm.at[wid], sidx)
        pltpu.sync_copy(dst_hbm.at[wid], didx)
        pltpu.sync_copy(gt_hbm.at[seg_sl], sh_g.at[seg_sl])
        pltpu.sync_copy(gt_hbm.at[seg_sl], sh_agg.at[seg_sl])
        plsc.subcore_barrier()

        def step(f, carry):
            col = pl.ds(f * N_PAD, N_PAD)
            pltpu.async_copy(sh_g.at[col].at[sidx], vals, sem).wait()
            pltpu.sync_copy(vals, sh_agg.at[col].at[didx], add=True)
            return carry

        lax.fori_loop(0, D, step, 0)
        plsc.subcore_barrier()
        pltpu.sync_copy(sh_agg.at[seg_sl], out_hbm.at[c, seg_sl])

    return sc_scatter


def _make_sc_scatter_grouped():
    D = 32
    NGF = 8
    FPG = 4
    EQ = E_PAD // 4
    CH = 10112
    NCHUNK = EQ // CH
    G16 = CH // 16
    SEG = D * N_PAD // NUM_SUBCORES
    AGGW = FPG * N_PAD
    HALF = AGGW // 2

    @functools.partial(
        pl.kernel,
        out_type=(jax.ShapeDtypeStruct((NUM_CORES * D * N_PAD,), _F32),
                  jax.ShapeDtypeStruct((NW * AGGW,), _F32)),
        mesh=_mesh(),
        compiler_params=pltpu.CompilerParams(needs_layout_passes=False),
        scratch_types=[
            pltpu.VMEM((CH,), jnp.int32),
            pltpu.VMEM((CH,), jnp.int32),
            pltpu.VMEM((CH,), _F32),
            pltpu.VMEM((CH,), _F32),
            pltpu.VMEM((CH,), _F32),
            pltpu.VMEM((CH,), _F32),
            pltpu.VMEM((AGGW,), _F32),
            pltpu.VMEM_SHARED((D * N_PAD,), _F32),
            pltpu.SemaphoreType.DMA,
            pltpu.SemaphoreType.DMA,
            pltpu.SemaphoreType.DMA,
            pltpu.SemaphoreType.DMA,
        ],
    )
    def sc_scatter(gt_hbm, srcq_hbm, dstq_hbm, out_hbm, stage_hbm,
                   sidx, didx, v0, v1, v2, v3, agg,
                   sh_g, s0, s1, s2, s3):
        vals = [v0, v1, v2, v3]
        sems = [s0, s1, s2, s3]
        c = lax.axis_index("c")
        s = lax.axis_index("s")
        q = s % NGF
        e = s // NGF
        quarter = c * 2 + e
        seg_sl = pl.ds(s * SEG, SEG)
        pltpu.sync_copy(gt_hbm.at[seg_sl], sh_g.at[seg_sl])

        def zf(j, carry):
            agg[pl.ds(pl.multiple_of(j * 16, 16), 16)] = jnp.zeros((16,), _F32)
            return carry

        @pl.when(e == 1)
        def _():
            lax.fori_loop(0, AGGW // 16, zf, 0)

        plsc.subcore_barrier()

        @pl.when(e == 0)
        def _():
            pltpu.sync_copy(sh_g.at[pl.ds(q * AGGW, AGGW)], agg)

        def chunk_body(t, carry):
            base = pl.multiple_of(quarter * EQ + t * CH, 128)
            pltpu.sync_copy(srcq_hbm.at[pl.ds(base, CH)], sidx)
            pltpu.sync_copy(dstq_hbm.at[pl.ds(base, CH)], didx)
            descs = [pltpu.async_copy(
                sh_g.at[pl.ds((q * FPG + j) * N_PAD, N_PAD)].at[sidx],
                vals[j], sems[j]) for j in range(FPG)]
            for j in range(FPG):
                descs[j].wait()

                def grp(g, carry2, _j=j):
                    sl = pl.ds(pl.multiple_of(g * 16, 16), 16)
                    tgt = didx[sl] + jnp.int32(_j * N_PAD)
                    plsc.addupdate_scatter(agg, [tgt], vals[_j][sl])
                    return carry2

                lax.fori_loop(0, G16, grp, 0)
            return carry

        lax.fori_loop(0, NCHUNK, chunk_body, 0)

        wbase = pl.multiple_of((c * NUM_SUBCORES + s) * AGGW, 128)
        pltpu.sync_copy(agg, stage_hbm.at[pl.ds(wbase, AGGW)])
        plsc.subcore_barrier()
        half = s // NGF
        moff = q * AGGW + half * HALF
        cbase = c * NUM_SUBCORES * AGGW
        pltpu.sync_copy(
            stage_hbm.at[pl.ds(pl.multiple_of(cbase + moff, 128), HALF)],
            agg.at[pl.ds(0, HALF)])
        pltpu.sync_copy(
            stage_hbm.at[pl.ds(
                pl.multiple_of(cbase + NGF * AGGW + moff, 128), HALF)],
            agg.at[pl.ds(HALF, HALF)])

        def addl(k, carry):
            a = pl.ds(pl.multiple_of(k * 16, 16), 16)
            b = pl.ds(pl.multiple_of(HALF + k * 16, 16), 16)
            agg[a] = agg[a] + agg[b]
            return carry

        lax.fori_loop(0, HALF // 16, addl, 0)
        pltpu.sync_copy(agg.at[pl.ds(0, HALF)],
                        out_hbm.at[pl.ds(
                            pl.multiple_of(c * D * N_PAD + moff, 128), HALF)])

    return sc_scatter


@functools.partial(
    pl.kernel,
    out_type=jax.ShapeDtypeStruct((NUM_CORES, N_PAD), _F32),
    mesh=_mesh(),
    scratch_types=[
        pltpu.VMEM((EW,), jnp.int32),
        pltpu.VMEM((EW,), _F32),
        pltpu.VMEM((ROWS_PER_TILE,), _F32),
        pltpu.VMEM_SHARED((N_PAD,), _F32),
    ],
)
def _sc_degree(dst_hbm, ones_hbm, out_hbm, didx, ones_v, zeros_v, sh_deg):
    c = lax.axis_index("c")
    s = lax.axis_index("s")
    wid = s * NUM_CORES + c
    row_sl = pl.ds(s * ROWS_PER_TILE, ROWS_PER_TILE)
    pltpu.sync_copy(dst_hbm.at[wid], didx)
    pltpu.sync_copy(ones_hbm, ones_v)

    def zfill(j, carry):
        zeros_v[pl.ds(j * 16, 16)] = jnp.zeros((16,), _F32)
        return carry

    lax.fori_loop(0, ROWS_PER_TILE // 16, zfill, 0)
    pltpu.sync_copy(zeros_v, sh_deg.at[row_sl])
    plsc.subcore_barrier()
    pltpu.sync_copy(ones_v, sh_deg.at[didx], add=True)
    plsc.subcore_barrier()
    pltpu.sync_copy(sh_deg.at[row_sl], out_hbm.at[c, row_sl])


def _tc1(deg2, x_pad, W1):
    def body(deg_ref, x_ref, w_ref, g_ref, dinv_ref):
        deg = deg_ref[pl.ds(0, 1)] + deg_ref[pl.ds(1, 1)] + 1.0
        dinv = lax.rsqrt(deg)
        h = lax.dot_general(w_ref[...], x_ref[...], (((0,), (1,)), ((), ())),
                            preferred_element_type=_F32, precision=_HIGH)
        g_ref[...] = dinv * h
        dinv_ref[...] = dinv

    return pl.pallas_call(
        body,
        out_shape=(jax.ShapeDtypeStruct((32, N_PAD), _F32),
                   jax.ShapeDtypeStruct((1, N_PAD), _F32)),
    )(deg2, x_pad, W1)


def _tc2(agg, gt, dinv, b2d):
    def body(agg_ref, g_ref, dinv_ref, b_ref, o_ref):
        a = agg_ref[0] + agg_ref[1] - g_ref[...]
        h = jax.nn.relu(dinv_ref[...] * a + b_ref[...])
        o_ref[...] = dinv_ref[...] * h

    return pl.pallas_call(
        body,
        out_shape=jax.ShapeDtypeStruct((32, N_PAD), _F32),
    )(agg, gt, dinv, b2d)


def _tc3(agg, gt, dinv, b2d, W2, W3):
    def body(agg_ref, g_ref, dinv_ref, b_ref, w2_ref, w3_ref, o_ref):
        s2 = dinv_ref[...] * (agg_ref[0] + agg_ref[1] - g_ref[...])
        h2 = jax.nn.relu(
            lax.dot_general(w2_ref[...], s2, (((0,), (0,)), ((), ())),
                            preferred_element_type=_F32, precision=_HIGH)
            + b_ref[...])
        o_ref[...] = dinv_ref[...] * lax.dot_general(
            w3_ref[...], h2, (((0,), (0,)), ((), ())),
            preferred_element_type=_F32, precision=_HIGH)

    return pl.pallas_call(
        body,
        out_shape=jax.ShapeDtypeStruct((16, N_PAD), _F32),
    )(agg, gt, dinv, b2d, W2, W3)


def _tc_final(agg, gt, dinv, b2d, batchT):
    def body(agg_ref, g_ref, dinv_ref, b_ref, batch_ref, o_ref):
        a = agg_ref[0] + agg_ref[1] - g_ref[...]
        out3 = dinv_ref[...] * a + b_ref[...]
        gid = lax.broadcasted_iota(jnp.int32, (NG, 1), 0)
        oh = (batch_ref[...] == gid).astype(_F32)
        sums = lax.dot_general(oh, out3, (((1,), (1,)), ((), ())),
                               preferred_element_type=_F32, precision=_HIGH)
        counts = jnp.sum(oh, axis=1, keepdims=True)
        pooled = sums / jnp.maximum(counts, 1.0)
        m = jnp.max(pooled, axis=1, keepdims=True)
        lse = jnp.log(jnp.sum(jnp.exp(pooled - m), axis=1, keepdims=True))
        o_ref[...] = pooled - m - lse

    return pl.pallas_call(
        body,
        out_shape=jax.ShapeDtypeStruct((NG, NC_OUT), _F32),
    )(agg, gt, dinv, b2d, batchT)


def kernel(x, edge_index, batch, W1, b1, W2, b2, W3, b3):
    src, dst = edge_index[0], edge_index[1]
    npad = E_PAD - E
    pad_ids = (jnp.arange(npad, dtype=jnp.int32) % (N_PAD - N)) + N
    src_p = jnp.concatenate([src, pad_ids]).reshape(NW, EW)
    dst_p = jnp.concatenate([dst, pad_ids]).reshape(NW, EW)

    x_pad = jnp.pad(x, ((0, N_PAD - N), (0, 0)))
    batchT = jnp.pad(batch, (0, N_PAD - N),
                     constant_values=NG).reshape(1, N_PAD)
    ones_e = jnp.ones((EW,), _F32)

    src_q = src_p.reshape(-1)
    dst_q = dst_p.reshape(-1)

    scat32 = _make_sc_scatter_grouped()
    deg = _sc_degree(dst_p, ones_e)
    g1, dinv = _tc1(deg, x_pad, W1)
    agg1, _ = scat32(g1.reshape(-1), src_q, dst_q)
    g2 = _tc2(agg1.reshape(2, 32, N_PAD), g1, dinv, b1.reshape(32, 1))
    agg2, _ = scat32(g2.reshape(-1), src_q, dst_q)
    g3 = _tc3(agg2.reshape(2, 32, N_PAD), g2, dinv,
              b2.reshape(64, 1), W2, W3)
    agg3 = _make_sc_scatter(16)(g3.reshape(-1), src_p, dst_p)
    return _tc_final(agg3.reshape(2, 16, N_PAD), g3, dinv,
                     b3.reshape(16, 1), batchT)


# speedup vs baseline: 1.0368x; 1.0368x over previous
import functools

import jax
import jax.numpy as jnp
from jax import lax
from jax.experimental import pallas as pl
from jax.experimental.pallas import tpu as pltpu
from jax.experimental.pallas import tpu_sc as plsc

N = 10000
E = 320000
F_IN = 128
NC_OUT = 16
NG = 64

N_PAD = 10240
NUM_CORES = 2
NUM_SUBCORES = 16
NW = NUM_CORES * NUM_SUBCORES
EW = 10112
E_PAD = NW * EW
ROWS_PER_TILE = N_PAD // NUM_SUBCORES

_F32 = jnp.float32
_HIGH = jax.lax.Precision.HIGHEST


def _mesh():
    return plsc.VectorSubcoreMesh(
        core_axis_name="c", subcore_axis_name="s",
        num_cores=NUM_CORES, num_subcores=NUM_SUBCORES)


def _make_sc_scatter(D):
    SEG = D * N_PAD // NUM_SUBCORES

    @functools.partial(
        pl.kernel,
        out_type=jax.ShapeDtypeStruct((NUM_CORES, D * N_PAD), _F32),
        mesh=_mesh(),
        scratch_types=[
            pltpu.VMEM((EW,), jnp.int32),
            pltpu.VMEM((EW,), jnp.int32),
            pltpu.VMEM((EW,), _F32),
            pltpu.VMEM_SHARED((D * N_PAD,), _F32),
            pltpu.VMEM_SHARED((D * N_PAD,), _F32),
            pltpu.SemaphoreType.DMA,
        ],
    )
    def sc_scatter(gt_hbm, src_hbm, dst_hbm, out_hbm,
                   sidx, didx, vals, sh_g, sh_agg, sem):
        c = lax.axis_index("c")
        s = lax.axis_index("s")
        wid = s * NUM_CORES + c
        seg_sl = pl.ds(s * SEG, SEG)
        pltpu.sync_copy(src_hbm.at[wid], sidx)
        pltpu.sync_copy(dst_hbm.at[wid], didx)
        pltpu.sync_copy(gt_hbm.at[seg_sl], sh_g.at[seg_sl])
        pltpu.sync_copy(gt_hbm.at[seg_sl], sh_agg.at[seg_sl])
        plsc.subcore_barrier()

        def step(f, carry):
            col = pl.ds(f * N_PAD, N_PAD)
            pltpu.async_copy(sh_g.at[col].at[sidx], vals, sem).wait()
            pltpu.sync_copy(vals, sh_agg.at[col].at[didx], add=True)
            return carry

        lax.fori_loop(0, D, step, 0)
        plsc.subcore_barrier()
        pltpu.sync_copy(sh_agg.at[seg_sl], out_hbm.at[c, seg_sl])

    return sc_scatter


def _make_sc_scatter_grouped():
    D = 32
    NGF = 8
    FPG = 4
    EQ = E_PAD // 4
    CH = 10112
    NCHUNK = EQ // CH
    G16 = CH // 16
    SEG = D * N_PAD // NUM_SUBCORES
    AGGW = FPG * N_PAD
    HALF = AGGW // 2

    @functools.partial(
        pl.kernel,
        out_type=(jax.ShapeDtypeStruct((NUM_CORES * D * N_PAD,), _F32),
                  jax.ShapeDtypeStruct((NW * AGGW,), _F32)),
        mesh=_mesh(),
        compiler_params=pltpu.CompilerParams(needs_layout_passes=False),
        scratch_types=[
            pltpu.VMEM((CH,), jnp.int32),
            pltpu.VMEM((CH,), jnp.int32),
            pltpu.VMEM((CH,), _F32),
            pltpu.VMEM((CH,), _F32),
            pltpu.VMEM((CH,), _F32),
            pltpu.VMEM((CH,), _F32),
            pltpu.VMEM((AGGW,), _F32),
            pltpu.VMEM_SHARED((D * N_PAD,), _F32),
            pltpu.SemaphoreType.DMA,
            pltpu.SemaphoreType.DMA,
            pltpu.SemaphoreType.DMA,
            pltpu.SemaphoreType.DMA,
        ],
    )
    def sc_scatter(gt_hbm, srcq_hbm, dstq_hbm, out_hbm, stage_hbm,
                   sidx, didx, v0, v1, v2, v3, agg,
                   sh_g, s0, s1, s2, s3):
        vals = [v0, v1, v2, v3]
        sems = [s0, s1, s2, s3]
        c = lax.axis_index("c")
        s = lax.axis_index("s")
        q = s % NGF
        e = s // NGF
        quarter = c * 2 + e
        seg_sl = pl.ds(s * SEG, SEG)
        pltpu.sync_copy(gt_hbm.at[seg_sl], sh_g.at[seg_sl])

        def zf(j, carry):
            agg[pl.ds(pl.multiple_of(j * 16, 16), 16)] = jnp.zeros((16,), _F32)
            return carry

        @pl.when(e == 1)
        def _():
            lax.fori_loop(0, AGGW // 16, zf, 0)

        plsc.subcore_barrier()

        @pl.when(e == 0)
        def _():
            pltpu.sync_copy(sh_g.at[pl.ds(q * AGGW, AGGW)], agg)

        def chunk_body(t, carry):
            base = pl.multiple_of(quarter * EQ + t * CH, 128)
            pltpu.sync_copy(srcq_hbm.at[pl.ds(base, CH)], sidx)
            pltpu.sync_copy(dstq_hbm.at[pl.ds(base, CH)], didx)
            descs = [pltpu.async_copy(
                sh_g.at[pl.ds((q * FPG + j) * N_PAD, N_PAD)].at[sidx],
                vals[j], sems[j]) for j in range(FPG)]
            for j in range(FPG):
                descs[j].wait()

                def grp(g, carry2, _j=j):
                    sl = pl.ds(pl.multiple_of(g * 16, 16), 16)
                    tgt = didx[sl] + jnp.int32(_j * N_PAD)
                    plsc.addupdate_scatter(agg, [tgt], vals[_j][sl])
                    return carry2

                lax.fori_loop(0, G16, grp, 0, unroll=8)
            return carry

        lax.fori_loop(0, NCHUNK, chunk_body, 0)

        wbase = pl.multiple_of((c * NUM_SUBCORES + s) * AGGW, 128)
        pltpu.sync_copy(agg, stage_hbm.at[pl.ds(wbase, AGGW)])
        plsc.subcore_barrier()
        half = s // NGF
        moff = q * AGGW + half * HALF
        cbase = c * NUM_SUBCORES * AGGW
        pltpu.sync_copy(
            stage_hbm.at[pl.ds(pl.multiple_of(cbase + moff, 128), HALF)],
            agg.at[pl.ds(0, HALF)])
        pltpu.sync_copy(
            stage_hbm.at[pl.ds(
                pl.multiple_of(cbase + NGF * AGGW + moff, 128), HALF)],
            agg.at[pl.ds(HALF, HALF)])

        def addl(k, carry):
            a = pl.ds(pl.multiple_of(k * 16, 16), 16)
            b = pl.ds(pl.multiple_of(HALF + k * 16, 16), 16)
            agg[a] = agg[a] + agg[b]
            return carry

        lax.fori_loop(0, HALF // 16, addl, 0, unroll=8)
        pltpu.sync_copy(agg.at[pl.ds(0, HALF)],
                        out_hbm.at[pl.ds(
                            pl.multiple_of(c * D * N_PAD + moff, 128), HALF)])

    return sc_scatter


@functools.partial(
    pl.kernel,
    out_type=jax.ShapeDtypeStruct((NUM_CORES, N_PAD), _F32),
    mesh=_mesh(),
    scratch_types=[
        pltpu.VMEM((EW,), jnp.int32),
        pltpu.VMEM((EW,), _F32),
        pltpu.VMEM((ROWS_PER_TILE,), _F32),
        pltpu.VMEM_SHARED((N_PAD,), _F32),
    ],
)
def _sc_degree(dst_hbm, ones_hbm, out_hbm, didx, ones_v, zeros_v, sh_deg):
    c = lax.axis_index("c")
    s = lax.axis_index("s")
    wid = s * NUM_CORES + c
    row_sl = pl.ds(s * ROWS_PER_TILE, ROWS_PER_TILE)
    pltpu.sync_copy(dst_hbm.at[wid], didx)
    pltpu.sync_copy(ones_hbm, ones_v)

    def zfill(j, carry):
        zeros_v[pl.ds(j * 16, 16)] = jnp.zeros((16,), _F32)
        return carry

    lax.fori_loop(0, ROWS_PER_TILE // 16, zfill, 0)
    pltpu.sync_copy(zeros_v, sh_deg.at[row_sl])
    plsc.subcore_barrier()
    pltpu.sync_copy(ones_v, sh_deg.at[didx], add=True)
    plsc.subcore_barrier()
    pltpu.sync_copy(sh_deg.at[row_sl], out_hbm.at[c, row_sl])


def _tc1(deg2, x_pad, W1):
    def body(deg_ref, x_ref, w_ref, g_ref, dinv_ref):
        deg = deg_ref[pl.ds(0, 1)] + deg_ref[pl.ds(1, 1)] + 1.0
        dinv = lax.rsqrt(deg)
        h = lax.dot_general(w_ref[...], x_ref[...], (((0,), (1,)), ((), ())),
                            preferred_element_type=_F32, precision=_HIGH)
        g_ref[...] = dinv * h
        dinv_ref[...] = dinv

    return pl.pallas_call(
        body,
        out_shape=(jax.ShapeDtypeStruct((32, N_PAD), _F32),
                   jax.ShapeDtypeStruct((1, N_PAD), _F32)),
    )(deg2, x_pad, W1)


def _tc2(agg, gt, dinv, b2d):
    def body(agg_ref, g_ref, dinv_ref, b_ref, o_ref):
        a = agg_ref[0] + agg_ref[1] - g_ref[...]
        h = jax.nn.relu(dinv_ref[...] * a + b_ref[...])
        o_ref[...] = dinv_ref[...] * h

    return pl.pallas_call(
        body,
        out_shape=jax.ShapeDtypeStruct((32, N_PAD), _F32),
    )(agg, gt, dinv, b2d)


def _tc3(agg, gt, dinv, b2d, W2, W3):
    def body(agg_ref, g_ref, dinv_ref, b_ref, w2_ref, w3_ref, o_ref):
        s2 = dinv_ref[...] * (agg_ref[0] + agg_ref[1] - g_ref[...])
        h2 = jax.nn.relu(
            lax.dot_general(w2_ref[...], s2, (((0,), (0,)), ((), ())),
                            preferred_element_type=_F32, precision=_HIGH)
            + b_ref[...])
        o_ref[...] = dinv_ref[...] * lax.dot_general(
            w3_ref[...], h2, (((0,), (0,)), ((), ())),
            preferred_element_type=_F32, precision=_HIGH)

    return pl.pallas_call(
        body,
        out_shape=jax.ShapeDtypeStruct((16, N_PAD), _F32),
    )(agg, gt, dinv, b2d, W2, W3)


def _tc_final(agg, gt, dinv, b2d, batchT):
    def body(agg_ref, g_ref, dinv_ref, b_ref, batch_ref, o_ref):
        a = agg_ref[0] + agg_ref[1] - g_ref[...]
        out3 = dinv_ref[...] * a + b_ref[...]
        gid = lax.broadcasted_iota(jnp.int32, (NG, 1), 0)
        oh = (batch_ref[...] == gid).astype(_F32)
        sums = lax.dot_general(oh, out3, (((1,), (1,)), ((), ())),
                               preferred_element_type=_F32, precision=_HIGH)
        counts = jnp.sum(oh, axis=1, keepdims=True)
        pooled = sums / jnp.maximum(counts, 1.0)
        m = jnp.max(pooled, axis=1, keepdims=True)
        lse = jnp.log(jnp.sum(jnp.exp(pooled - m), axis=1, keepdims=True))
        o_ref[...] = pooled - m - lse

    return pl.pallas_call(
        body,
        out_shape=jax.ShapeDtypeStruct((NG, NC_OUT), _F32),
    )(agg, gt, dinv, b2d, batchT)


def kernel(x, edge_index, batch, W1, b1, W2, b2, W3, b3):
    src, dst = edge_index[0], edge_index[1]
    npad = E_PAD - E
    pad_ids = (jnp.arange(npad, dtype=jnp.int32) % (N_PAD - N)) + N
    src_p = jnp.concatenate([src, pad_ids]).reshape(NW, EW)
    dst_p = jnp.concatenate([dst, pad_ids]).reshape(NW, EW)

    x_pad = jnp.pad(x, ((0, N_PAD - N), (0, 0)))
    batchT = jnp.pad(batch, (0, N_PAD - N),
                     constant_values=NG).reshape(1, N_PAD)
    ones_e = jnp.ones((EW,), _F32)

    src_q = src_p.reshape(-1)
    dst_q = dst_p.reshape(-1)

    scat32 = _make_sc_scatter_grouped()
    deg = _sc_degree(dst_p, ones_e)
    g1, dinv = _tc1(deg, x_pad, W1)
    agg1, _ = scat32(g1.reshape(-1), src_q, dst_q)
    g2 = _tc2(agg1.reshape(2, 32, N_PAD), g1, dinv, b1.reshape(32, 1))
    agg2, _ = scat32(g2.reshape(-1), src_q, dst_q)
    g3 = _tc3(agg2.reshape(2, 32, N_PAD), g2, dinv,
              b2.reshape(64, 1), W2, W3)
    agg3 = _make_sc_scatter(16)(g3.reshape(-1), src_p, dst_p)
    return _tc_final(agg3.reshape(2, 16, N_PAD), g3, dinv,
                     b3.reshape(16, 1), batchT)
